# baseline (speedup 1.0000x reference)
import functools

import jax
import jax.numpy as jnp
from jax import lax
from jax.experimental import pallas as pl
from jax.experimental.pallas import tpu as pltpu
from jax.experimental.pallas import tpu_sc as plsc


def _silu(x):
    return x * jax.nn.sigmoid(x)


def _proj_kernel(h_ref, wr_ref, wc_ref, pr_ref, pc_ref):
    hb = h_ref[...]
    pr_ref[...] = jnp.dot(hb, wr_ref[...], preferred_element_type=jnp.float32)
    pc_ref[...] = jnp.dot(hb, wc_ref[...], preferred_element_type=jnp.float32)


def _edge_kernel(gsum_ref, crow_ref, ccol_ref, ea_ref,
                 wg_ref, wea_ref, b1_ref, w2_ref, b2_ref, lng_ref, lnb_ref,
                 cw1_ref, cb1_ref, cw2_ref, ef_ref, tr_ref):
    eps = 1e-8
    thr = 1e-6
    rx = crow_ref[:, 0:1]
    ry = crow_ref[:, 1:2]
    rz = crow_ref[:, 2:3]
    cx = ccol_ref[:, 0:1]
    cy = ccol_ref[:, 1:2]
    cz = ccol_ref[:, 2:3]
    d0 = rx - cx
    d1 = ry - cy
    d2 = rz - cz
    radial = d0 * d0 + d1 * d1 + d2 * d2
    dist = jnp.sqrt(radial)
    dot = rx * cx + ry * cy + rz * cz
    ainv = 1.0 / (dist + eps)
    a0 = d0 * ainv
    a1 = d1 * ainv
    a2 = d2 * ainv
    u0 = ry * cz - rz * cy
    u1 = rz * cx - rx * cz
    u2 = rx * cy - ry * cx
    un = jnp.sqrt(u0 * u0 + u1 * u1 + u2 * u2)
    binv = 1.0 / (un + eps)
    b0 = u0 * binv
    b1v = u1 * binv
    b2v = u2 * binv
    c0 = a1 * b2v - a2 * b1v
    c1 = a2 * b0 - a0 * b2v
    c2 = a0 * b1v - a1 * b0
    na = jnp.sqrt(a0 * a0 + a1 * a1 + a2 * a2)
    nb = jnp.sqrt(b0 * b0 + b1v * b1v + b2v * b2v)
    nc = jnp.sqrt(c0 * c0 + c1 * c1 + c2 * c2)
    mask = (na < thr) | (nb < thr) | (nc < thr)
    one = jnp.ones_like(a0)
    zero = jnp.zeros_like(a0)
    so3 = [jnp.where(mask, idv, v) for v, idv in
           ((a0, one), (b0, zero), (c0, zero),
            (a1, zero), (b1v, one), (c1, zero),
            (a2, zero), (b2v, zero), (c2, one))]
    geo = [radial, dist, dot] + so3
    pre = gsum_ref[...] + b1_ref[...]
    wg = wg_ref[...]
    for k, gk in enumerate(geo):
        pre = pre + gk * wg[k:k + 1, :]
    pre = pre + jnp.dot(ea_ref[...], wea_ref[...],
                        preferred_element_type=jnp.float32)
    sp = _silu(pre)
    t = jnp.dot(sp, w2_ref[...], preferred_element_type=jnp.float32) + b2_ref[...]
    mu = jnp.mean(t, axis=1, keepdims=True)
    var = jnp.mean((t - mu) * (t - mu), axis=1, keepdims=True)
    ef = lng_ref[...] * (t - mu) * jax.lax.rsqrt(var + 1e-5) + lnb_ref[...]
    ef_ref[...] = ef
    uu = _silu(jnp.dot(ef, cw1_ref[...], preferred_element_type=jnp.float32)
               + cb1_ref[...])
    cm = jnp.sum(uu * cw2_ref[...], axis=1, keepdims=True)
    z = jnp.zeros((d0.shape[0], 13), dtype=cm.dtype)
    tr_ref[...] = jnp.concatenate([d0 * cm, d1 * cm, d2 * cm, z], axis=1)


def _node_kernel(h_ref, agg0_ref, agg1_ref, cp_ref, cagg0_ref, cagg1_ref,
                 w1h_ref, w1a_ref, nb1_ref, nw2_ref, nb2_ref, hn_ref, cn_ref):
    hb = h_ref[...]
    agg = agg0_ref[...] + agg1_ref[...]
    pre = (jnp.dot(hb, w1h_ref[...], preferred_element_type=jnp.float32)
           + jnp.dot(agg, w1a_ref[...], preferred_element_type=jnp.float32)
           + nb1_ref[...])
    o = jnp.dot(_silu(pre), nw2_ref[...], preferred_element_type=jnp.float32) \
        + nb2_ref[...]
    hn_ref[...] = hb + o
    cn_ref[...] = cp_ref[...] + cagg0_ref[:, :16] + cagg1_ref[:, :16]


_NC = 2
_NS = 16
_NW = _NC * _NS
_CH = 80


def _make_sc_gather(N, E, HID):
    ept = E // _NW
    nchunk = ept // _CH
    f32 = jnp.float32
    i32 = jnp.int32
    mesh = plsc.VectorSubcoreMesh(core_axis_name="c", subcore_axis_name="s")

    @functools.partial(
        pl.kernel, mesh=mesh,
        out_type=[jax.ShapeDtypeStruct((E, HID), f32),
                  jax.ShapeDtypeStruct((E, 16), f32),
                  jax.ShapeDtypeStruct((E, 16), f32)],
        scratch_types=[pltpu.VMEM((nchunk, _CH), i32),
                       pltpu.VMEM((nchunk, _CH), i32),
                       pltpu.VMEM((_CH, HID), f32),
                       pltpu.VMEM((_CH, HID), f32),
                       pltpu.VMEM((_CH, HID), f32),
                       pltpu.VMEM((_CH, HID), f32),
                       pltpu.VMEM((_CH, 16), f32),
                       pltpu.VMEM((_CH, 16), f32),
                       pltpu.SemaphoreType.DMA,
                       pltpu.SemaphoreType.DMA,
                       pltpu.SemaphoreType.DMA,
                       pltpu.SemaphoreType.DMA],
    )
    def gk(prow_hbm, pcol_hbm, coordp_hbm, row3_hbm, col3_hbm,
           gsum_hbm, crow_hbm, ccol_hbm,
           idxr, idxc, bufr, bufc, cbufr, cbufc, packr, packc,
           s0, s1, s2, s3):
        cid = lax.axis_index("c")
        sid = lax.axis_index("s")
        wid = sid * _NC + cid
        base = wid * ept
        pltpu.sync_copy(row3_hbm.at[wid], idxr)
        pltpu.sync_copy(col3_hbm.at[wid], idxc)

        def body(j, carry):
            cr = pltpu.async_copy(prow_hbm.at[idxr.at[j]], bufr, s0)
            cc = pltpu.async_copy(pcol_hbm.at[idxc.at[j]], bufc, s1)
            c2 = pltpu.async_copy(coordp_hbm.at[idxr.at[j]], cbufr, s2)
            c3 = pltpu.async_copy(coordp_hbm.at[idxc.at[j]], cbufc, s3)
            c2.wait()
            c3.wait()

            def crow_body(r, c4):
                sl = pl.ds(0, 16)
                packr[r, sl] = cbufr[r, sl]
                packc[r, sl] = cbufc[r, sl]
                return c4

            lax.fori_loop(0, _CH, crow_body, 0)
            cr.wait()
            cc.wait()

            def addrow(r, c5):
                for kk in range(HID // 16):
                    sl = pl.ds(kk * 16, 16)
                    bufr[r, sl] = bufr[r, sl] + bufc[r, sl]
                return c5

            lax.fori_loop(0, _CH, addrow, 0)
            off = base + j * _CH
            pltpu.sync_copy(bufr, gsum_hbm.at[pl.ds(off, _CH)])
            pltpu.sync_copy(packr, crow_hbm.at[pl.ds(off, _CH)])
            pltpu.sync_copy(packc, ccol_hbm.at[pl.ds(off, _CH)])
            return carry

        lax.fori_loop(0, nchunk, body, 0)

    return gk


def _make_sc_scatter(NP, E, HID):
    ept = E // _NW
    nchunk = ept // _CH
    npt = NP // _NS
    f32 = jnp.float32
    mesh = plsc.VectorSubcoreMesh(core_axis_name="c", subcore_axis_name="s")

    @functools.partial(
        pl.kernel, mesh=mesh,
        out_type=[jax.ShapeDtypeStruct((_NC, NP, HID), f32),
                  jax.ShapeDtypeStruct((_NC, NP, HID), f32)],
        scratch_types=[pltpu.VMEM((nchunk, _CH), jnp.int32),
                       pltpu.VMEM((_CH, HID), f32),
                       pltpu.VMEM((_CH, HID), f32),
                       pltpu.VMEM((_CH, 16), f32),
                       pltpu.VMEM_SHARED((NP, HID), f32)],
    )
    def sk(ef_hbm, tr_hbm, row3_hbm, zh_hbm, agg_hbm, cagg_hbm,
           idxr, buf, wide, trb, aggs):
        cid = lax.axis_index("c")
        sid = lax.axis_index("s")
        wid = sid * _NC + cid
        base = wid * ept
        myrows = pl.ds(sid * npt, npt)
        pltpu.sync_copy(row3_hbm.at[wid], idxr)
        pltpu.sync_copy(zh_hbm.at[myrows], aggs.at[myrows])
        pltpu.sync_copy(zh_hbm.at[pl.ds(0, _CH)], wide)
        plsc.subcore_barrier()

        def body1(j, carry):
            off = base + j * _CH
            pltpu.sync_copy(ef_hbm.at[pl.ds(off, _CH)], buf)
            pltpu.sync_copy(buf, aggs.at[idxr.at[j]], add=True)
            return carry

        lax.fori_loop(0, nchunk, body1, 0)
        plsc.subcore_barrier()
        pltpu.sync_copy(aggs.at[myrows], agg_hbm.at[cid, myrows])
        pltpu.sync_copy(zh_hbm.at[myrows], aggs.at[myrows])
        plsc.subcore_barrier()

        def body2(j, carry):
            off = base + j * _CH
            pltpu.sync_copy(tr_hbm.at[pl.ds(off, _CH)], trb)

            def expand(r, c2):
                sl = pl.ds(0, 16)
                wide[r, sl] = trb[r, sl]
                return c2

            lax.fori_loop(0, _CH, expand, 0)
            pltpu.sync_copy(wide, aggs.at[idxr.at[j]], add=True)
            return carry

        lax.fori_loop(0, nchunk, body2, 0)
        plsc.subcore_barrier()
        pltpu.sync_copy(aggs.at[myrows], cagg_hbm.at[cid, myrows])

    return sk


def kernel(h, edge_index, coord, edge_attr, eW1, eb1, eW2, eb2, ln_g, ln_b,
           nW1, nb1, nW2, nb2, cW1, cb1, cW2):
    N, D = h.shape
    E = edge_index.shape[1]
    NH, FEAT, HD = eW1.shape
    HID = NH * HD
    f32 = jnp.float32

    row = edge_index[0]
    col = edge_index[1]

    W1full = jnp.transpose(eW1, (1, 0, 2)).reshape(FEAT, HID)
    Wr = W1full[:D]
    Wc = W1full[D:2 * D]
    Wg = jnp.zeros((16, HID), f32).at[:12].set(W1full[2 * D:2 * D + 12])
    Wea = W1full[2 * D + 12:]
    b1 = eb1.reshape(1, HID)
    W2bd = jnp.zeros((HID, HID), f32)
    for i in range(NH):
        W2bd = W2bd.at[i * HD:(i + 1) * HD, i * HD:(i + 1) * HD].set(eW2[i])
    b2 = eb2.reshape(1, HID)
    lng = ln_g.reshape(1, HID)
    lnb = ln_b.reshape(1, HID)
    cb1r = cb1.reshape(1, HID)
    cw2r = cW2.reshape(1, HID)
    nb1r = nb1.reshape(1, HID)
    nb2r = nb2.reshape(1, D)
    nW1h = nW1[:D]
    nW1a = nW1[D:]

    coordp16 = jnp.zeros((N, 16), f32).at[:, :3].set(coord)
    coordp = jnp.zeros((N, HID), f32).at[:, :3].set(coord)

    BN = 1000
    pr, pc = pl.pallas_call(
        _proj_kernel,
        grid=(N // BN,),
        in_specs=[
            pl.BlockSpec((BN, D), lambda i: (i, 0)),
            pl.BlockSpec((D, HID), lambda i: (0, 0)),
            pl.BlockSpec((D, HID), lambda i: (0, 0)),
        ],
        out_specs=[
            pl.BlockSpec((BN, HID), lambda i: (i, 0)),
            pl.BlockSpec((BN, HID), lambda i: (i, 0)),
        ],
        out_shape=[
            jax.ShapeDtypeStruct((N, HID), f32),
            jax.ShapeDtypeStruct((N, HID), f32),
        ],
    )(h, Wr, Wc)

    ept = E // _NW
    row3 = row.reshape(_NW, ept // _CH, _CH)
    col3 = col.reshape(_NW, ept // _CH, _CH)
    gsum, crow, ccol = _make_sc_gather(N, E, HID)(
        pr, pc, coordp, row3, col3)

    BE = 2000
    wspec = lambda shape: pl.BlockSpec(shape, lambda i: (0, 0))
    ef, tr = pl.pallas_call(
        _edge_kernel,
        grid=(E // BE,),
        in_specs=[
            pl.BlockSpec((BE, HID), lambda i: (i, 0)),
            pl.BlockSpec((BE, 16), lambda i: (i, 0)),
            pl.BlockSpec((BE, 16), lambda i: (i, 0)),
            pl.BlockSpec((BE, 16), lambda i: (i, 0)),
            wspec((16, HID)),
            wspec((16, HID)),
            wspec((1, HID)),
            wspec((HID, HID)),
            wspec((1, HID)),
            wspec((1, HID)),
            wspec((1, HID)),
            wspec((HID, HID)),
            wspec((1, HID)),
            wspec((1, HID)),
        ],
        out_specs=[
            pl.BlockSpec((BE, HID), lambda i: (i, 0)),
            pl.BlockSpec((BE, 16), lambda i: (i, 0)),
        ],
        out_shape=[
            jax.ShapeDtypeStruct((E, HID), f32),
            jax.ShapeDtypeStruct((E, 16), f32),
        ],
    )(gsum, crow, ccol, edge_attr,
      Wg, Wea, b1, W2bd, b2, lng, lnb, cW1, cb1r, cw2r)

    NP = ((N + 127) // 128) * 128
    zh = jnp.zeros((NP, HID), f32)
    aggp, caggp = _make_sc_scatter(NP, E, HID)(ef, tr, row3, zh)

    hn, cn = pl.pallas_call(
        _node_kernel,
        grid=(N // BN,),
        in_specs=[
            pl.BlockSpec((BN, D), lambda i: (i, 0)),
            pl.BlockSpec((BN, HID), lambda i: (i, 0)),
            pl.BlockSpec((BN, HID), lambda i: (i, 0)),
            pl.BlockSpec((BN, 16), lambda i: (i, 0)),
            pl.BlockSpec((BN, HID), lambda i: (i, 0)),
            pl.BlockSpec((BN, HID), lambda i: (i, 0)),
            wspec((D, HID)),
            wspec((HID, HID)),
            wspec((1, HID)),
            wspec((HID, D)),
            wspec((1, D)),
        ],
        out_specs=[
            pl.BlockSpec((BN, D), lambda i: (i, 0)),
            pl.BlockSpec((BN, 16), lambda i: (i, 0)),
        ],
        out_shape=[
            jax.ShapeDtypeStruct((N, D), f32),
            jax.ShapeDtypeStruct((N, 16), f32),
        ],
    )(h, aggp[0], aggp[1], coordp16, caggp[0], caggp[1],
      nW1h, nW1a, nb1r, nW2, nb2r)

    return (hn, cn[:, :3], edge_attr)


# speedup vs baseline: 1.0463x; 1.0463x over previous
import functools

import jax
import jax.numpy as jnp
from jax import lax
from jax.experimental import pallas as pl
from jax.experimental.pallas import tpu as pltpu
from jax.experimental.pallas import tpu_sc as plsc


def _silu(x):
    return x * jax.nn.sigmoid(x)


def _proj_kernel(h_ref, wr_ref, wc_ref, pr_ref, pc_ref):
    hb = h_ref[...]
    pr_ref[...] = jnp.dot(hb, wr_ref[...], preferred_element_type=jnp.float32)
    pc_ref[...] = jnp.dot(hb, wc_ref[...], preferred_element_type=jnp.float32)


def _edge_kernel(gsum_ref, crow_ref, ccol_ref, ea_ref,
                 wg_ref, wea_ref, b1_ref, w2_ref, b2_ref, lng_ref, lnb_ref,
                 cw1_ref, cb1_ref, cw2_ref, ef_ref, tr_ref):
    eps = 1e-8
    thr = 1e-6
    rx = crow_ref[:, 0:1]
    ry = crow_ref[:, 1:2]
    rz = crow_ref[:, 2:3]
    cx = ccol_ref[:, 0:1]
    cy = ccol_ref[:, 1:2]
    cz = ccol_ref[:, 2:3]
    d0 = rx - cx
    d1 = ry - cy
    d2 = rz - cz
    radial = d0 * d0 + d1 * d1 + d2 * d2
    dist = jnp.sqrt(radial)
    dot = rx * cx + ry * cy + rz * cz
    ainv = 1.0 / (dist + eps)
    a0 = d0 * ainv
    a1 = d1 * ainv
    a2 = d2 * ainv
    u0 = ry * cz - rz * cy
    u1 = rz * cx - rx * cz
    u2 = rx * cy - ry * cx
    un = jnp.sqrt(u0 * u0 + u1 * u1 + u2 * u2)
    binv = 1.0 / (un + eps)
    b0 = u0 * binv
    b1v = u1 * binv
    b2v = u2 * binv
    c0 = a1 * b2v - a2 * b1v
    c1 = a2 * b0 - a0 * b2v
    c2 = a0 * b1v - a1 * b0
    na = jnp.sqrt(a0 * a0 + a1 * a1 + a2 * a2)
    nb = jnp.sqrt(b0 * b0 + b1v * b1v + b2v * b2v)
    nc = jnp.sqrt(c0 * c0 + c1 * c1 + c2 * c2)
    mask = (na < thr) | (nb < thr) | (nc < thr)
    one = jnp.ones_like(a0)
    zero = jnp.zeros_like(a0)
    so3 = [jnp.where(mask, idv, v) for v, idv in
           ((a0, one), (b0, zero), (c0, zero),
            (a1, zero), (b1v, one), (c1, zero),
            (a2, zero), (b2v, zero), (c2, one))]
    geo = [radial, dist, dot] + so3
    pre = gsum_ref[...] + b1_ref[...]
    wg = wg_ref[...]
    for k, gk in enumerate(geo):
        pre = pre + gk * wg[k:k + 1, :]
    pre = pre + jnp.dot(ea_ref[...], wea_ref[...],
                        preferred_element_type=jnp.float32)
    sp = _silu(pre)
    t = jnp.dot(sp, w2_ref[...], preferred_element_type=jnp.float32) + b2_ref[...]
    mu = jnp.mean(t, axis=1, keepdims=True)
    var = jnp.mean((t - mu) * (t - mu), axis=1, keepdims=True)
    ef = lng_ref[...] * (t - mu) * jax.lax.rsqrt(var + 1e-5) + lnb_ref[...]
    ef_ref[...] = ef
    uu = _silu(jnp.dot(ef, cw1_ref[...], preferred_element_type=jnp.float32)
               + cb1_ref[...])
    cm = jnp.sum(uu * cw2_ref[...], axis=1, keepdims=True)
    z = jnp.zeros((d0.shape[0], 13), dtype=cm.dtype)
    tr_ref[...] = jnp.concatenate([d0 * cm, d1 * cm, d2 * cm, z], axis=1)


def _node_kernel(h_ref, agg0_ref, agg1_ref, cp_ref, cagg0_ref, cagg1_ref,
                 w1h_ref, w1a_ref, nb1_ref, nw2_ref, nb2_ref, hn_ref, cn_ref):
    hb = h_ref[...]
    agg = agg0_ref[...] + agg1_ref[...]
    pre = (jnp.dot(hb, w1h_ref[...], preferred_element_type=jnp.float32)
           + jnp.dot(agg, w1a_ref[...], preferred_element_type=jnp.float32)
           + nb1_ref[...])
    o = jnp.dot(_silu(pre), nw2_ref[...], preferred_element_type=jnp.float32) \
        + nb2_ref[...]
    hn_ref[...] = hb + o
    cn_ref[...] = cp_ref[...] + cagg0_ref[:, :16] + cagg1_ref[:, :16]


_NC = 2
_NS = 16
_NW = _NC * _NS
_CHG = 40
_CHS = 80


def _make_sc_gather(N, E, HID):
    ept = E // _NW
    nchunk = ept // _CHG
    f32 = jnp.float32
    i32 = jnp.int32
    mesh = plsc.VectorSubcoreMesh(core_axis_name="c", subcore_axis_name="s")

    @functools.partial(
        pl.kernel, mesh=mesh,
        out_type=[jax.ShapeDtypeStruct((E, HID), f32),
                  jax.ShapeDtypeStruct((E, 16), f32),
                  jax.ShapeDtypeStruct((E, 16), f32)],
        scratch_types=[pltpu.VMEM((nchunk, _CHG), i32),
                       pltpu.VMEM((nchunk, _CHG), i32),
                       pltpu.VMEM((2, _CHG, HID), f32),
                       pltpu.VMEM((2, _CHG, HID), f32),
                       pltpu.VMEM((2, _CHG, HID), f32),
                       pltpu.VMEM((2, _CHG, HID), f32),
                       pltpu.VMEM((_CHG, 16), f32),
                       pltpu.VMEM((_CHG, 16), f32),
                       pltpu.SemaphoreType.DMA,
                       pltpu.SemaphoreType.DMA],
    )
    def gk(prow_hbm, pcol_hbm, coordp_hbm, row3_hbm, col3_hbm,
           gsum_hbm, crow_hbm, ccol_hbm,
           idxr, idxc, bufr, bufc, cbufr, cbufc, packr, packc,
           sg0, sg1):
        cid = lax.axis_index("c")
        sid = lax.axis_index("s")
        wid = sid * _NC + cid
        base = wid * ept
        pltpu.sync_copy(row3_hbm.at[wid], idxr)
        pltpu.sync_copy(col3_hbm.at[wid], idxc)
        sems = (sg0, sg1)

        def issue(j, b):
            sem = sems[b]
            pltpu.async_copy(prow_hbm.at[idxr.at[j]], bufr.at[b], sem)
            pltpu.async_copy(pcol_hbm.at[idxc.at[j]], bufc.at[b], sem)
            pltpu.async_copy(coordp_hbm.at[idxr.at[j]], cbufr.at[b], sem)
            pltpu.async_copy(coordp_hbm.at[idxc.at[j]], cbufc.at[b], sem)

        def process(j, b):
            sem = sems[b]
            pltpu.make_async_copy(prow_hbm.at[idxr.at[j]], bufr.at[b], sem).wait()
            pltpu.make_async_copy(pcol_hbm.at[idxc.at[j]], bufc.at[b], sem).wait()
            pltpu.make_async_copy(coordp_hbm.at[idxr.at[j]], cbufr.at[b], sem).wait()
            pltpu.make_async_copy(coordp_hbm.at[idxc.at[j]], cbufc.at[b], sem).wait()

            def crow_body(r, c4):
                sl = pl.ds(0, 16)
                packr[r, sl] = cbufr[b, r, sl]
                packc[r, sl] = cbufc[b, r, sl]
                return c4

            lax.fori_loop(0, _CHG, crow_body, 0)

            def addrow(r, c5):
                for kk in range(HID // 16):
                    sl = pl.ds(kk * 16, 16)
                    bufr[b, r, sl] = bufr[b, r, sl] + bufc[b, r, sl]
                return c5

            lax.fori_loop(0, _CHG, addrow, 0)
            off = base + j * _CHG
            pltpu.sync_copy(bufr.at[b], gsum_hbm.at[pl.ds(off, _CHG)])
            pltpu.sync_copy(packr, crow_hbm.at[pl.ds(off, _CHG)])
            pltpu.sync_copy(packc, ccol_hbm.at[pl.ds(off, _CHG)])

        issue(0, 0)
        issue(1, 1)

        def body(m, carry):
            for b in range(2):
                j = 2 * m + b
                process(j, b)
                jn = j + 2

                @pl.when(jn < nchunk)
                def _():
                    issue(jn, b)

            return carry

        lax.fori_loop(0, nchunk // 2, body, 0)
        if nchunk % 2:
            process(nchunk - 1, (nchunk - 1) % 2)

    return gk


def _make_sc_scatter_ef(NP, E, HID):
    ept = E // _NW
    nchunk = ept // _CHS
    npt = NP // _NS
    f32 = jnp.float32
    mesh = plsc.VectorSubcoreMesh(core_axis_name="c", subcore_axis_name="s")

    @functools.partial(
        pl.kernel, mesh=mesh,
        out_type=jax.ShapeDtypeStruct((_NC, NP, HID), f32),
        scratch_types=[pltpu.VMEM((nchunk, _CHS), jnp.int32),
                       pltpu.VMEM((2, _CHS, HID), f32),
                       pltpu.VMEM_SHARED((NP, HID), f32),
                       pltpu.SemaphoreType.DMA,
                       pltpu.SemaphoreType.DMA],
    )
    def sk(ef_hbm, row3_hbm, zh_hbm, agg_hbm, idxr, buf, aggs, sl0, sl1):
        cid = lax.axis_index("c")
        sid = lax.axis_index("s")
        wid = sid * _NC + cid
        base = wid * ept
        myrows = pl.ds(sid * npt, npt)
        sems = (sl0, sl1)
        pltpu.sync_copy(row3_hbm.at[wid], idxr)
        pltpu.sync_copy(zh_hbm.at[myrows], aggs.at[myrows])
        plsc.subcore_barrier()

        def ld1(j, b):
            pltpu.async_copy(ef_hbm.at[pl.ds(base + j * _CHS, _CHS)],
                             buf.at[b], sems[b])

        ld1(0, 0)

        def step1(j, b):
            pltpu.make_async_copy(ef_hbm.at[pl.ds(base + j * _CHS, _CHS)],
                                  buf.at[b], sems[b]).wait()
            jn = j + 1

            @pl.when(jn < nchunk)
            def _():
                ld1(jn, 1 - b)

            pltpu.sync_copy(buf.at[b], aggs.at[idxr.at[j]], add=True)

        def body1(m, carry):
            for b in range(2):
                step1(2 * m + b, b)
            return carry

        lax.fori_loop(0, nchunk // 2, body1, 0)
        if nchunk % 2:
            step1(nchunk - 1, 0)
        plsc.subcore_barrier()
        pltpu.sync_copy(aggs.at[myrows], agg_hbm.at[cid, myrows])

    return sk


def _make_sc_scatter_tr(NP, E, HID):
    ept = E // _NW
    nchunk = ept // _CHS
    npt = NP // _NS
    f32 = jnp.float32
    mesh = plsc.VectorSubcoreMesh(core_axis_name="c", subcore_axis_name="s")

    @functools.partial(
        pl.kernel, mesh=mesh,
        out_type=jax.ShapeDtypeStruct((_NC, NP, HID), f32),
        scratch_types=[pltpu.VMEM((nchunk, _CHS), jnp.int32),
                       pltpu.VMEM((_CHS, 16), f32),
                       pltpu.VMEM((_CHS, HID), f32),
                       pltpu.VMEM_SHARED((NP, HID), f32)],
    )
    def sk(tr_hbm, row3_hbm, zh_hbm, cagg_hbm, idxr, trb, wide, aggs):
        cid = lax.axis_index("c")
        sid = lax.axis_index("s")
        wid = sid * _NC + cid
        base = wid * ept
        myrows = pl.ds(sid * npt, npt)
        pltpu.sync_copy(row3_hbm.at[wid], idxr)
        pltpu.sync_copy(zh_hbm.at[myrows], aggs.at[myrows])
        pltpu.sync_copy(zh_hbm.at[pl.ds(0, _CHS)], wide)
        plsc.subcore_barrier()

        def body2(j, carry):
            off = base + j * _CHS
            pltpu.sync_copy(tr_hbm.at[pl.ds(off, _CHS)], trb)

            def expand(r, c2):
                sl = pl.ds(0, 16)
                wide[r, sl] = trb[r, sl]
                return c2

            lax.fori_loop(0, _CHS, expand, 0)
            pltpu.sync_copy(wide, aggs.at[idxr.at[j]], add=True)
            return carry

        lax.fori_loop(0, nchunk, body2, 0)
        plsc.subcore_barrier()
        pltpu.sync_copy(aggs.at[myrows], cagg_hbm.at[cid, myrows])

    return sk


def kernel(h, edge_index, coord, edge_attr, eW1, eb1, eW2, eb2, ln_g, ln_b,
           nW1, nb1, nW2, nb2, cW1, cb1, cW2):
    N, D = h.shape
    E = edge_index.shape[1]
    NH, FEAT, HD = eW1.shape
    HID = NH * HD
    f32 = jnp.float32

    row = edge_index[0]
    col = edge_index[1]

    W1full = jnp.transpose(eW1, (1, 0, 2)).reshape(FEAT, HID)
    Wr = W1full[:D]
    Wc = W1full[D:2 * D]
    Wg = jnp.zeros((16, HID), f32).at[:12].set(W1full[2 * D:2 * D + 12])
    Wea = W1full[2 * D + 12:]
    b1 = eb1.reshape(1, HID)
    W2bd = jnp.zeros((HID, HID), f32)
    for i in range(NH):
        W2bd = W2bd.at[i * HD:(i + 1) * HD, i * HD:(i + 1) * HD].set(eW2[i])
    b2 = eb2.reshape(1, HID)
    lng = ln_g.reshape(1, HID)
    lnb = ln_b.reshape(1, HID)
    cb1r = cb1.reshape(1, HID)
    cw2r = cW2.reshape(1, HID)
    nb1r = nb1.reshape(1, HID)
    nb2r = nb2.reshape(1, D)
    nW1h = nW1[:D]
    nW1a = nW1[D:]

    coordp16 = jnp.zeros((N, 16), f32).at[:, :3].set(coord)
    coordp = jnp.zeros((N, HID), f32).at[:, :3].set(coord)

    BN = 1000
    pr, pc = pl.pallas_call(
        _proj_kernel,
        grid=(N // BN,),
        in_specs=[
            pl.BlockSpec((BN, D), lambda i: (i, 0)),
            pl.BlockSpec((D, HID), lambda i: (0, 0)),
            pl.BlockSpec((D, HID), lambda i: (0, 0)),
        ],
        out_specs=[
            pl.BlockSpec((BN, HID), lambda i: (i, 0)),
            pl.BlockSpec((BN, HID), lambda i: (i, 0)),
        ],
        out_shape=[
            jax.ShapeDtypeStruct((N, HID), f32),
            jax.ShapeDtypeStruct((N, HID), f32),
        ],
    )(h, Wr, Wc)

    ept = E // _NW
    row3g = row.reshape(_NW, ept // _CHG, _CHG)
    col3g = col.reshape(_NW, ept // _CHG, _CHG)
    row3s = row.reshape(_NW, ept // _CHS, _CHS)
    gsum, crow, ccol = _make_sc_gather(N, E, HID)(
        pr, pc, coordp, row3g, col3g)

    BE = 2000
    wspec = lambda shape: pl.BlockSpec(shape, lambda i: (0, 0))
    ef, tr = pl.pallas_call(
        _edge_kernel,
        grid=(E // BE,),
        in_specs=[
            pl.BlockSpec((BE, HID), lambda i: (i, 0)),
            pl.BlockSpec((BE, 16), lambda i: (i, 0)),
            pl.BlockSpec((BE, 16), lambda i: (i, 0)),
            pl.BlockSpec((BE, 16), lambda i: (i, 0)),
            wspec((16, HID)),
            wspec((16, HID)),
            wspec((1, HID)),
            wspec((HID, HID)),
            wspec((1, HID)),
            wspec((1, HID)),
            wspec((1, HID)),
            wspec((HID, HID)),
            wspec((1, HID)),
            wspec((1, HID)),
        ],
        out_specs=[
            pl.BlockSpec((BE, HID), lambda i: (i, 0)),
            pl.BlockSpec((BE, 16), lambda i: (i, 0)),
        ],
        out_shape=[
            jax.ShapeDtypeStruct((E, HID), f32),
            jax.ShapeDtypeStruct((E, 16), f32),
        ],
    )(gsum, crow, ccol, edge_attr,
      Wg, Wea, b1, W2bd, b2, lng, lnb, cW1, cb1r, cw2r)

    NP = ((N + 127) // 128) * 128
    zh = jnp.zeros((NP, HID), f32)
    aggp = _make_sc_scatter_ef(NP, E, HID)(ef, row3s, zh)
    caggp = _make_sc_scatter_tr(NP, E, HID)(tr, row3s, zh)

    hn, cn = pl.pallas_call(
        _node_kernel,
        grid=(N // BN,),
        in_specs=[
            pl.BlockSpec((BN, D), lambda i: (i, 0)),
            pl.BlockSpec((BN, HID), lambda i: (i, 0)),
            pl.BlockSpec((BN, HID), lambda i: (i, 0)),
            pl.BlockSpec((BN, 16), lambda i: (i, 0)),
            pl.BlockSpec((BN, HID), lambda i: (i, 0)),
            pl.BlockSpec((BN, HID), lambda i: (i, 0)),
            wspec((D, HID)),
            wspec((HID, HID)),
            wspec((1, HID)),
            wspec((HID, D)),
            wspec((1, D)),
        ],
        out_specs=[
            pl.BlockSpec((BN, D), lambda i: (i, 0)),
            pl.BlockSpec((BN, 16), lambda i: (i, 0)),
        ],
        out_shape=[
            jax.ShapeDtypeStruct((N, D), f32),
            jax.ShapeDtypeStruct((N, 16), f32),
        ],
    )(h, aggp[0], aggp[1], coordp16, caggp[0], caggp[1],
      nW1h, nW1a, nb1r, nW2, nb2r)

    return (hn, cn[:, :3], edge_attr)
